# baseline (speedup 1.0000x reference)
import jax
import jax.numpy as jnp
from jax.experimental import pallas as pl
from jax.experimental.pallas import tpu as pltpu

TAU = 0.7

BN = 8192

_INV_SQRT2 = 0.7071067811865476


def _gelu(v):
    return 0.5 * v * (1.0 + jax.lax.erf(v * _INV_SQRT2))


def _fused_kernel(x_ref, w1_ref, b1_ref, w2_ref, b2_ref, mk_ref, mvT_ref,
                  wh1aT_ref, wh1bT_ref, bh1_ref, wh2T_ref, bh2_ref, out_ref):
    x = x_ref[...]
    z1 = _gelu(jnp.dot(x, w1_ref[...]) + b1_ref[...])
    z = _gelu(jnp.dot(z1, w2_ref[...]) + b2_ref[...])

    zT = z.T
    logits = jnp.dot(mk_ref[...], zT) * (1.0 / TAU)

    v0 = jnp.max(logits, axis=0, keepdims=True)
    eq0 = (logits == v0).astype(jnp.float32)
    masked = logits - eq0 * jnp.float32(1e30)
    v1 = jnp.max(masked, axis=0, keepdims=True)
    eq1 = (masked == v1).astype(jnp.float32)

    e = jnp.exp(v1 - v0)
    denom = 1.0 + e
    p0 = 1.0 / (denom * jnp.sum(eq0, axis=0, keepdims=True))
    p1 = e / (denom * jnp.sum(eq1, axis=0, keepdims=True))
    attnT = p0 * eq0 + p1 * eq1
    memT = jnp.dot(mvT_ref[...], attnT)

    hT = _gelu(jnp.dot(wh1aT_ref[...], zT) + jnp.dot(wh1bT_ref[...], memT)
               + bh1_ref[...])
    head = jnp.dot(wh2T_ref[...], hT) + bh2_ref[...]
    out_ref[...] = jax.nn.sigmoid(head)[None]


@jax.jit
def _run(x, W1, b1, W2, b2, memory_keys, memory_values, Wh1, bh1, Wh2, bh2):
    n, d = x.shape
    h0 = W1.shape[1]
    l = W2.shape[1]
    k = memory_keys.shape[0]

    wh1aT = Wh1[:l].T
    wh1bT = Wh1[l:].T

    rep = lambda *shape: pl.BlockSpec(shape, lambda i: (0,) * len(shape))
    out = pl.pallas_call(
        _fused_kernel,
        grid=(n // BN,),
        in_specs=[
            pl.BlockSpec((BN, d), lambda i: (i, 0)),
            rep(d, h0), rep(1, h0), rep(h0, l), rep(1, l),
            rep(k, l), rep(l, k),
            rep(h0, l), rep(h0, l), rep(h0, 1), rep(1, h0), rep(1, 1),
        ],
        out_specs=pl.BlockSpec((1, 1, BN), lambda i: (i, 0, 0)),
        out_shape=jax.ShapeDtypeStruct((n // BN, 1, BN), jnp.float32),
        compiler_params=pltpu.CompilerParams(
            dimension_semantics=("parallel",),
            vmem_limit_bytes=110 * 1024 * 1024),
    )(x, W1, b1.reshape(1, -1), W2, b2.reshape(1, -1), memory_keys,
      memory_values.T, wh1aT, wh1bT, bh1.reshape(-1, 1), Wh2.T,
      bh2.reshape(1, 1))
    return out.reshape(n)


def kernel(x, W1, b1, W2, b2, memory_keys, memory_values, Wh1, bh1, Wh2, bh2):
    return _run(x, W1, b1, W2, b2, memory_keys, memory_values,
                Wh1, bh1, Wh2, bh2)


# speedup vs baseline: 1.0332x; 1.0332x over previous
import jax
import jax.numpy as jnp
from jax.experimental import pallas as pl
from jax.experimental.pallas import tpu as pltpu

TAU = 0.7

BN = 4096

_INV_SQRT2 = 0.7071067811865476


def _gelu(v):
    return 0.5 * v * (1.0 + jax.lax.erf(v * _INV_SQRT2))


def _fused_kernel(x_ref, w1_ref, b1_ref, w2_ref, b2_ref, mk_ref, mvT_ref,
                  wh1aT_ref, wh1bT_ref, bh1_ref, wh2T_ref, bh2_ref, out_ref):
    x = x_ref[...]
    z1 = _gelu(jnp.dot(x, w1_ref[...]) + b1_ref[...])
    z = _gelu(jnp.dot(z1, w2_ref[...]) + b2_ref[...])

    zT = z.T
    logits = jnp.dot(mk_ref[...], zT) * (1.0 / TAU)

    v0 = jnp.max(logits, axis=0, keepdims=True)
    eq0 = (logits == v0).astype(jnp.float32)
    masked = logits - eq0 * jnp.float32(1e30)
    v1 = jnp.max(masked, axis=0, keepdims=True)
    eq1 = (masked == v1).astype(jnp.float32)

    e = jnp.exp(v1 - v0)
    denom = 1.0 + e
    p0 = 1.0 / (denom * jnp.sum(eq0, axis=0, keepdims=True))
    p1 = e / (denom * jnp.sum(eq1, axis=0, keepdims=True))
    attnT = p0 * eq0 + p1 * eq1
    memT = jnp.dot(mvT_ref[...], attnT)

    hT = _gelu(jnp.dot(wh1aT_ref[...], zT) + jnp.dot(wh1bT_ref[...], memT)
               + bh1_ref[...])
    head = jnp.dot(wh2T_ref[...], hT) + bh2_ref[...]
    out_ref[...] = jax.nn.sigmoid(head)[None]


@jax.jit
def _run(x, W1, b1, W2, b2, memory_keys, memory_values, Wh1, bh1, Wh2, bh2):
    n, d = x.shape
    h0 = W1.shape[1]
    l = W2.shape[1]
    k = memory_keys.shape[0]

    wh1aT = Wh1[:l].T
    wh1bT = Wh1[l:].T

    rep = lambda *shape: pl.BlockSpec(shape, lambda i: (0,) * len(shape))
    out = pl.pallas_call(
        _fused_kernel,
        grid=(n // BN,),
        in_specs=[
            pl.BlockSpec((BN, d), lambda i: (i, 0)),
            rep(d, h0), rep(1, h0), rep(h0, l), rep(1, l),
            rep(k, l), rep(l, k),
            rep(h0, l), rep(h0, l), rep(h0, 1), rep(1, h0), rep(1, 1),
        ],
        out_specs=pl.BlockSpec((1, 1, BN), lambda i: (i, 0, 0)),
        out_shape=jax.ShapeDtypeStruct((n // BN, 1, BN), jnp.float32),
        compiler_params=pltpu.CompilerParams(
            dimension_semantics=("parallel",),
            vmem_limit_bytes=110 * 1024 * 1024),
    )(x, W1, b1.reshape(1, -1), W2, b2.reshape(1, -1), memory_keys,
      memory_values.T, wh1aT, wh1bT, bh1.reshape(-1, 1), Wh2.T,
      bh2.reshape(1, 1))
    return out.reshape(n)


def kernel(x, W1, b1, W2, b2, memory_keys, memory_values, Wh1, bh1, Wh2, bh2):
    return _run(x, W1, b1, W2, b2, memory_keys, memory_values,
                Wh1, bh1, Wh2, bh2)
